# baseline (speedup 1.0000x reference)
import functools

import jax
import jax.numpy as jnp
from jax import lax
from jax.experimental import pallas as pl
from jax.experimental.pallas import tpu as pltpu
from jax.experimental.pallas import tpu_sc as plsc

B = 16384
K = 64
N_JK = 151
JB_PAD = 160
NC, NS, L = 2, 16, 16
NW = NC * NS
BPW = B // NW
CH = 128
NCH = BPW // CH
BLK_PER_CH = CH // L

_mesh = plsc.VectorSubcoreMesh(core_axis_name="c", subcore_axis_name="s")


@functools.partial(
    pl.kernel,
    mesh=_mesh,
    out_type=jax.ShapeDtypeStruct((B,), jnp.float32),
    compiler_params=pltpu.CompilerParams(
        needs_layout_passes=False, use_tc_tiling_on_sc=False),
    scratch_types=[
        pltpu.VMEM((BPW,), jnp.int32),
        pltpu.VMEM((BPW,), jnp.int32),
        pltpu.VMEM((BPW, K), jnp.float32),
        pltpu.VMEM((BPW,), jnp.float32),
        pltpu.VMEM((N_JK, K), jnp.float32),
        pltpu.VMEM((JB_PAD,), jnp.float32),
        pltpu.VMEM((L,), jnp.float32),
        pltpu.VMEM((BPW,), jnp.float32),
    ] + [pltpu.SemaphoreType.DMA] * (2 * NCH),
)
def _sc_dot(users_hbm, jokes_hbm, ulut_hbm, jlut_hbm, ubias_hbm, jbias_hbm,
            gb_hbm, out_hbm,
            uidx_v, jdx_v, urows_v, ub_v, jlut_v, jb_v, gb_v, out_v,
            *sems):
    wid = lax.axis_index("s") * NC + lax.axis_index("c")
    base = wid * BPW

    pltpu.sync_copy(users_hbm.at[pl.ds(base, BPW)], uidx_v)

    row_cps, bias_cps = [], []
    for c in range(NCH):
        idx = uidx_v.at[pl.ds(c * CH, CH)]
        row_cps.append(pltpu.async_copy(
            ulut_hbm.at[idx], urows_v.at[pl.ds(c * CH, CH)], sems[c]))
        bias_cps.append(pltpu.async_copy(
            ubias_hbm.at[idx], ub_v.at[pl.ds(c * CH, CH)], sems[NCH + c]))

    pltpu.sync_copy(jokes_hbm.at[pl.ds(base, BPW)], jdx_v)
    pltpu.sync_copy(jlut_hbm, jlut_v)
    pltpu.sync_copy(jbias_hbm, jb_v)
    pltpu.sync_copy(gb_hbm, gb_v)

    lane = lax.iota(jnp.int32, L)
    zero16 = jnp.zeros((L,), jnp.int32)
    gbv = gb_v[...]

    def block(b, carry):
        b0 = pl.multiple_of(b * L, L)
        jvec = jdx_v[pl.ds(b0, L)]
        q = jnp.zeros((L,), jnp.float32)
        for i in range(L):
            r = b0 + i
            jr = jvec[i]
            t = urows_v[r, pl.ds(0, L)] * jlut_v[jr, pl.ds(0, L)]
            t = urows_v[r, pl.ds(L, L)] * jlut_v[jr, pl.ds(L, L)] + t
            t = urows_v[r, pl.ds(2 * L, L)] * jlut_v[jr, pl.ds(2 * L, L)] + t
            t = urows_v[r, pl.ds(3 * L, L)] * jlut_v[jr, pl.ds(3 * L, L)] + t
            s = lax.reduce_sum(t, axes=(0,))
            q = jnp.where(lane == i, s, q)
        jbv = plsc.load_gather(jb_v, [jvec])
        out_v[pl.ds(b0, L)] = q + ub_v[pl.ds(b0, L)] + jbv + gbv
        return carry

    for c in range(NCH):
        row_cps[c].wait()
        bias_cps[c].wait()
        lax.fori_loop(c * BLK_PER_CH, (c + 1) * BLK_PER_CH, block, 0,
                      unroll=True)

    pltpu.sync_copy(out_v, out_hbm.at[pl.ds(base, BPW)])


def kernel(users, jokes, user_lut, joke_lut, user_bias, joke_bias, global_bias):
    ub = user_bias.reshape(-1)
    jb = jnp.pad(joke_bias.reshape(-1), (0, JB_PAD - N_JK))
    gb = jnp.broadcast_to(global_bias, (L,))
    return _sc_dot(users, jokes, user_lut, joke_lut, ub, jb, gb)


# speedup vs baseline: 1.1106x; 1.1106x over previous
import functools

import jax
import jax.numpy as jnp
from jax import lax
from jax.experimental import pallas as pl
from jax.experimental.pallas import tpu as pltpu
from jax.experimental.pallas import tpu_sc as plsc

B = 16384
K = 64
N_JK = 151
JB_PAD = 160
NC, NS, L = 2, 16, 16
NW = NC * NS
BPW = B // NW
CH = 128
NCH = BPW // CH
BLK_PER_CH = CH // L

_mesh = plsc.VectorSubcoreMesh(core_axis_name="c", subcore_axis_name="s")


@functools.partial(
    pl.kernel,
    mesh=_mesh,
    out_type=jax.ShapeDtypeStruct((B,), jnp.float32),
    compiler_params=pltpu.CompilerParams(
        needs_layout_passes=False, use_tc_tiling_on_sc=False),
    scratch_types=[
        pltpu.VMEM((BPW,), jnp.int32),
        pltpu.VMEM((BPW,), jnp.int32),
        pltpu.VMEM((BPW, K), jnp.float32),
        pltpu.VMEM((BPW,), jnp.float32),
        pltpu.VMEM((N_JK, K), jnp.float32),
        pltpu.VMEM((JB_PAD,), jnp.float32),
        pltpu.VMEM((L,), jnp.float32),
        pltpu.VMEM((BPW,), jnp.float32),
    ] + [pltpu.SemaphoreType.DMA] * (2 * NCH),
)
def _sc_dot(users_hbm, jokes_hbm, ulut_hbm, jlut_hbm, ubias_hbm, jbias_hbm,
            gb_hbm, out_hbm,
            uidx_v, jdx_v, urows_v, ub_v, jlut_v, jb_v, gb_v, out_v,
            *sems):
    wid = lax.axis_index("s") * NC + lax.axis_index("c")
    base = wid * BPW

    pltpu.sync_copy(users_hbm.at[pl.ds(base, BPW)], uidx_v)

    row_cps, bias_cps = [], []
    for c in range(NCH):
        idx = uidx_v.at[pl.ds(c * CH, CH)]
        row_cps.append(pltpu.async_copy(
            ulut_hbm.at[idx], urows_v.at[pl.ds(c * CH, CH)], sems[c]))
        bias_cps.append(pltpu.async_copy(
            ubias_hbm.at[idx], ub_v.at[pl.ds(c * CH, CH)], sems[NCH + c]))

    pltpu.sync_copy(jokes_hbm.at[pl.ds(base, BPW)], jdx_v)
    pltpu.sync_copy(jlut_hbm, jlut_v)
    pltpu.sync_copy(jbias_hbm, jb_v)
    pltpu.sync_copy(gb_hbm, gb_v)

    lane = lax.iota(jnp.int32, L)
    zero16 = jnp.zeros((L,), jnp.int32)
    gbv = gb_v[...]

    def block(b, carry):
        b0 = pl.multiple_of(b * L, L)
        jvec = jdx_v[pl.ds(b0, L)]
        q = jnp.zeros((L,), jnp.float32)
        for i in range(L):
            r = b0 + i
            jr = jvec[i]
            t = urows_v[r, pl.ds(0, L)] * jlut_v[jr, pl.ds(0, L)]
            t = urows_v[r, pl.ds(L, L)] * jlut_v[jr, pl.ds(L, L)] + t
            t = urows_v[r, pl.ds(2 * L, L)] * jlut_v[jr, pl.ds(2 * L, L)] + t
            t = urows_v[r, pl.ds(3 * L, L)] * jlut_v[jr, pl.ds(3 * L, L)] + t
            s = lax.reduce_sum(t, axes=(0,))
            q = jnp.where(lane == i, s, q)
        jbv = plsc.load_gather(jb_v, [jvec])
        out_v[pl.ds(b0, L)] = q + ub_v[pl.ds(b0, L)] + jbv + gbv
        return carry

    for c in range(NCH):
        row_cps[c].wait()
        bias_cps[c].wait()
        lax.fori_loop(c * BLK_PER_CH, (c + 1) * BLK_PER_CH, block, 0,
                      unroll=False)

    pltpu.sync_copy(out_v, out_hbm.at[pl.ds(base, BPW)])


def kernel(users, jokes, user_lut, joke_lut, user_bias, joke_bias, global_bias):
    ub = user_bias.reshape(-1)
    jb = jnp.pad(joke_bias.reshape(-1), (0, JB_PAD - N_JK))
    gb = jnp.broadcast_to(global_bias, (L,))
    return _sc_dot(users, jokes, user_lut, joke_lut, ub, jb, gb)


# speedup vs baseline: 1.1293x; 1.0168x over previous
import functools

import jax
import jax.numpy as jnp
from jax import lax
from jax.experimental import pallas as pl
from jax.experimental.pallas import tpu as pltpu
from jax.experimental.pallas import tpu_sc as plsc

B = 16384
K = 64
N_JK = 151
JB_PAD = 160
NC, NS, L = 2, 16, 16
NW = NC * NS
BPW = B // NW
CH = 128
NCH = BPW // CH
BLK_PER_CH = CH // L

_mesh = plsc.VectorSubcoreMesh(core_axis_name="c", subcore_axis_name="s")


@functools.partial(
    pl.kernel,
    mesh=_mesh,
    out_type=jax.ShapeDtypeStruct((B,), jnp.float32),
    compiler_params=pltpu.CompilerParams(
        needs_layout_passes=False, use_tc_tiling_on_sc=False),
    scratch_types=[
        pltpu.VMEM((BPW,), jnp.int32),
        pltpu.VMEM((BPW,), jnp.int32),
        pltpu.VMEM((BPW, K), jnp.float32),
        pltpu.VMEM((BPW,), jnp.float32),
        pltpu.VMEM((N_JK, K), jnp.float32),
        pltpu.VMEM((JB_PAD, 1), jnp.float32),
        pltpu.VMEM((L,), jnp.float32),
        pltpu.VMEM((BPW,), jnp.float32),
    ] + [pltpu.SemaphoreType.DMA] * (2 * NCH),
)
def _sc_dot(users_hbm, jokes_hbm, ulut_hbm, jlut_hbm, ubias_hbm, jbias_hbm,
            gb_hbm, out_hbm,
            uidx_v, jdx_v, urows_v, ub_v, jlut_v, jb_v, gb_v, out_v,
            *sems):
    wid = lax.axis_index("s") * NC + lax.axis_index("c")
    base = wid * BPW

    pltpu.sync_copy(users_hbm.at[pl.ds(base, BPW)], uidx_v)

    row_cps, bias_cps = [], []
    for c in range(NCH):
        idx = uidx_v.at[pl.ds(c * CH, CH)]
        row_cps.append(pltpu.async_copy(
            ulut_hbm.at[idx], urows_v.at[pl.ds(c * CH, CH)], sems[c]))
        bias_cps.append(pltpu.async_copy(
            ubias_hbm.at[idx], ub_v.at[pl.ds(c * CH, CH)], sems[NCH + c]))

    pltpu.sync_copy(jokes_hbm.at[pl.ds(base, BPW)], jdx_v)
    pltpu.sync_copy(jlut_hbm, jlut_v)
    pltpu.sync_copy(jbias_hbm, jb_v.at[pl.ds(0, N_JK)])
    pltpu.sync_copy(gb_hbm, gb_v.at[pl.ds(0, 1)])

    lane = lax.iota(jnp.int32, L)
    zero16 = jnp.zeros((L,), jnp.int32)
    gbs = gb_v[...][0]

    def block(b, carry):
        b0 = pl.multiple_of(b * L, L)
        jvec = jdx_v[pl.ds(b0, L)]
        q = jnp.zeros((L,), jnp.float32)
        for i in range(L):
            r = b0 + i
            jr = jvec[i]
            t = urows_v[r, pl.ds(0, L)] * jlut_v[jr, pl.ds(0, L)]
            t = urows_v[r, pl.ds(L, L)] * jlut_v[jr, pl.ds(L, L)] + t
            t = urows_v[r, pl.ds(2 * L, L)] * jlut_v[jr, pl.ds(2 * L, L)] + t
            t = urows_v[r, pl.ds(3 * L, L)] * jlut_v[jr, pl.ds(3 * L, L)] + t
            s = lax.reduce_sum(t, axes=(0,))
            q = jnp.where(lane == i, s, q)
        jbv = plsc.load_gather(jb_v, [jvec, zero16])
        out_v[pl.ds(b0, L)] = q + ub_v[pl.ds(b0, L)] + jbv + gbs
        return carry

    for c in range(NCH):
        row_cps[c].wait()
        bias_cps[c].wait()
        lax.fori_loop(c * BLK_PER_CH, (c + 1) * BLK_PER_CH, block, 0,
                      unroll=False)

    pltpu.sync_copy(out_v, out_hbm.at[pl.ds(base, BPW)])


def kernel(users, jokes, user_lut, joke_lut, user_bias, joke_bias, global_bias):
    return _sc_dot(users, jokes, user_lut, joke_lut,
                   user_bias.reshape(-1), joke_bias, global_bias)
